# baseline (speedup 1.0000x reference)
import functools

import jax
import jax.numpy as jnp
from jax import lax
from jax.experimental import pallas as pl
from jax.experimental.pallas import tpu as pltpu
from jax.experimental.pallas import tpu_sc as plsc

N = 5000
R = 5
E = 64000
D_IN = 128
M = 64
OUT = 128
NC = 2
NS = 16

E_PAD = 65536
EW = E_PAD // NS
CW = 4
NCHUNK = EW // (CW * 128)
TAB = R * N
NPASS = R
AGG_ROWS = 5120
TRASH = 5000
PER_W = AGG_ROWS // NS
HIST_N = 5120
HIST_PW = HIST_N // NS

def _sc_mesh():
    return plsc.VectorSubcoreMesh(core_axis_name="c", subcore_axis_name="s")


def _deg_body(sidx_hbm, out_hbm, idx_v, ones_v, zb_v, hist_s):
    cid = lax.axis_index("c")
    sid = lax.axis_index("s")

    def _fill_z(i, _):
        zb_v[pl.ds(i * 16, 16)] = jnp.zeros((16,), jnp.float32)
        return 0

    lax.fori_loop(0, HIST_PW // 16, _fill_z, 0)

    def _fill_o(i, _):
        ones_v[pl.ds(i * 16, 16)] = jnp.ones((16,), jnp.float32)
        return 0

    lax.fori_loop(0, (CW * 128) // 16, _fill_o, 0)

    pltpu.sync_copy(zb_v, hist_s.at[pl.ds(sid * HIST_PW, HIST_PW)])
    plsc.subcore_barrier()

    for r in range(R):
        pltpu.sync_copy(sidx_hbm.at[cid, r, sid], idx_v)
        for c in range(NCHUNK):
            pltpu.sync_copy(ones_v, hist_s.at[idx_v.at[c]], add=True)

    plsc.subcore_barrier()
    pltpu.sync_copy(hist_s.at[pl.ds(sid * HIST_PW, HIST_PW)], zb_v)
    pltpu.sync_copy(zb_v, out_hbm.at[pl.ds(cid * HIST_N + sid * HIST_PW, HIST_PW)])


@functools.cache
def _deg_kernel():
    return pl.kernel(
        _deg_body,
        out_type=jax.ShapeDtypeStruct((NC * HIST_N,), jnp.float32),
        mesh=_sc_mesh(),
        compiler_params=pltpu.CompilerParams(use_tc_tiling_on_sc=False),
        scratch_types=[
            pltpu.VMEM((NCHUNK, CW * 128), jnp.int32),
            pltpu.VMEM((CW * 128,), jnp.float32),
            pltpu.VMEM((HIST_PW,), jnp.float32),
            pltpu.VMEM_SHARED((HIST_N,), jnp.float32),
        ],
    )


NTOT = R * NCHUNK


def _agg_body(tab_hbm, gidx_hbm, sidx_hbm, out_hbm,
              gi_v, si_v, rows_a, rows_b, rows_c, agg_s, sem_a, sem_b, sem_c):
    cid = lax.axis_index("c")
    sid = lax.axis_index("s")
    ring = (rows_a, rows_b, rows_c)
    sems = (sem_a, sem_b, sem_c)
    CR = CW * 128

    def _fill_z(i, _):
        rows_a[i // 4, pl.ds((i % 4) * 16, 16)] = jnp.zeros((16,), jnp.float32)
        return 0

    for p in range(NPASS):
        lax.fori_loop(0, PER_W * 4, _fill_z, 0)
        pltpu.sync_copy(rows_a.at[pl.ds(0, PER_W)],
                        agg_s.at[pl.ds(sid * PER_W, PER_W)])
        plsc.subcore_barrier()

        r = p
        pltpu.sync_copy(gidx_hbm.at[cid, r, sid], gi_v)
        pltpu.sync_copy(sidx_hbm.at[cid, r, sid], si_v)

        pend = [None, None, None]
        for j in range(NCHUNK):
            b = j % 3
            if pend[b] is not None:
                pend[b].wait()
                pend[b] = None
            g = pltpu.async_copy(tab_hbm.at[gi_v.at[j]], ring[b], sems[b])
            g.wait()
            pend[b] = pltpu.async_copy(ring[b], agg_s.at[si_v.at[j]],
                                       sems[b], add=True)
        for b in range(3):
            if pend[b] is not None:
                pend[b].wait()
                pend[b] = None

        plsc.subcore_barrier()
        base = (cid * NPASS + p) * AGG_ROWS + sid * PER_W
        pltpu.sync_copy(agg_s.at[pl.ds(sid * PER_W, PER_W)],
                        rows_a.at[pl.ds(0, PER_W)])
        pltpu.sync_copy(rows_a.at[pl.ds(0, PER_W)],
                        out_hbm.at[pl.ds(base, PER_W)])
        if p < NPASS - 1:
            plsc.subcore_barrier()


@functools.cache
def _agg_kernel():
    return pl.kernel(
        _agg_body,
        out_type=jax.ShapeDtypeStruct((NC * NPASS * AGG_ROWS, M), jnp.float32),
        mesh=_sc_mesh(),
        compiler_params=pltpu.CompilerParams(use_tc_tiling_on_sc=False),
        scratch_types=[
            pltpu.VMEM((NCHUNK, CW * 128), jnp.int32),
            pltpu.VMEM((NCHUNK, CW * 128), jnp.int32),
            pltpu.VMEM((CW * 128, M), jnp.float32),
            pltpu.VMEM((CW * 128, M), jnp.float32),
            pltpu.VMEM((CW * 128, M), jnp.float32),
            pltpu.VMEM_SHARED((AGG_ROWS, M), jnp.float32),
            pltpu.SemaphoreType.DMA,
            pltpu.SemaphoreType.DMA,
            pltpu.SemaphoreType.DMA,
        ],
    )


def _proj_body(feats_ref, w_ref, deg_ref, out_ref):
    c = lax.rsqrt(jnp.maximum(deg_ref[0, 0, :N], 1.0))
    out_ref[0] = (
        jnp.dot(feats_ref[0], w_ref[0], preferred_element_type=jnp.float32)
        * c[:, None]
    )


def _project(feats_s, w_all, deg_sw):
    return pl.pallas_call(
        _proj_body,
        grid=(2, R),
        in_specs=[
            pl.BlockSpec((1, N, D_IN), lambda d, r: (d, 0, 0)),
            pl.BlockSpec((1, D_IN, M), lambda d, r: (r, 0, 0)),
            pl.BlockSpec((1, 1, HIST_N), lambda d, r: (d, 0, 0)),
        ],
        out_specs=pl.BlockSpec((1, N, M), lambda d, r: (d * R + r, 0, 0)),
        out_shape=jax.ShapeDtypeStruct((2 * R, N, M), jnp.float32),
    )(feats_s, w_all, deg_sw)


def _out_body(agg_ref, deg_ref, fcw_ref, fcb_ref, out_ref):
    r = pl.program_id(1)
    c = lax.rsqrt(jnp.maximum(deg_ref[0, 0, :N], 1.0))
    x = jnp.maximum(agg_ref[0, 0] * c[:, None], 0.0)
    y = jnp.dot(x, fcw_ref[...], preferred_element_type=jnp.float32)

    @pl.when(r == 0)
    def _():
        out_ref[0] = y + fcb_ref[...]

    @pl.when(r > 0)
    def _():
        out_ref[0] += y


def _fc_out(agg, deg, fc_W, fc_b):
    return pl.pallas_call(
        _out_body,
        grid=(2, R),
        in_specs=[
            pl.BlockSpec((1, 1, N, M), lambda d, r: (d, r, 0, 0)),
            pl.BlockSpec((1, 1, HIST_N), lambda d, r: (d, 0, 0)),
            pl.BlockSpec((M, OUT), lambda d, r: (r, 0)),
            pl.BlockSpec((1, OUT), lambda d, r: (0, 0)),
        ],
        out_specs=pl.BlockSpec((1, N, OUT), lambda d, r: (d, 0, 0)),
        out_shape=jax.ShapeDtypeStruct((2, N, OUT), jnp.float32),
    )(agg, deg, fc_W, fc_b.reshape(1, OUT))


def kernel(user_feat, item_feat, edge_index, W_r, fc_W, fc_b):
    src = edge_index[:, 0, :].astype(jnp.int32)
    dst = edge_index[:, 1, :].astype(jnp.int32)

    npad = E_PAD - E
    lane = jnp.arange(npad, dtype=jnp.int32)
    pad_gath = jnp.broadcast_to(lane % 64, (R, npad))
    pad_scat = jnp.broadcast_to(TRASH + (lane % 16), (R, npad))

    roff = (jnp.arange(R, dtype=jnp.int32) * N)[:, None]
    g0 = jnp.concatenate([dst + roff, pad_gath], axis=1)
    s0 = jnp.concatenate([src, pad_scat], axis=1)
    g1 = jnp.concatenate([src + roff + TAB, pad_gath], axis=1)
    s1 = jnp.concatenate([dst, pad_scat], axis=1)
    gidx = jnp.stack([g0, g1]).reshape(NC, R, NS, NCHUNK, CW * 128)
    sidx = jnp.stack([s0, s1]).reshape(NC, R, NS, NCHUNK, CW * 128)

    deg = _deg_kernel()(sidx).reshape(NC, HIST_N)

    feats_s = jnp.stack([item_feat, user_feat])
    deg_sw = deg[::-1]
    tab = _project(feats_s, W_r, deg_sw.reshape(NC, 1, HIST_N)).reshape(2 * TAB, M)

    agg = _agg_kernel()(tab, gidx, sidx).reshape(NC, NPASS, AGG_ROWS, M)

    out = _fc_out(agg, deg.reshape(NC, 1, HIST_N), fc_W, fc_b)
    return out[0], out[1]


# speedup vs baseline: 1.0472x; 1.0472x over previous
import functools

import jax
import jax.numpy as jnp
from jax import lax
from jax.experimental import pallas as pl
from jax.experimental.pallas import tpu as pltpu
from jax.experimental.pallas import tpu_sc as plsc

N = 5000
R = 5
E = 64000
D_IN = 128
M = 64
OUT = 128
NC = 2
NS = 16

E_PAD = 65536
EW = E_PAD // NS
CW = 4
NCHUNK = EW // (CW * 128)
TAB = R * N
NPASS = 3
AGG_ROWS = 10240
TRASH = 10000
PER_W = AGG_ROWS // NS
ZROWS = 128
HIST_N = 10240
HIST_PW = HIST_N // NS

def _sc_mesh():
    return plsc.VectorSubcoreMesh(core_axis_name="c", subcore_axis_name="s")


def _deg_body(sidx_hbm, out_hbm, idx_v, ones_v, zb_v, hist_s):
    cid = lax.axis_index("c")
    sid = lax.axis_index("s")

    def _fill_z(i, _):
        zb_v[pl.ds(i * 16, 16)] = jnp.zeros((16,), jnp.float32)
        return 0

    lax.fori_loop(0, HIST_PW // 16, _fill_z, 0)

    def _fill_o(i, _):
        ones_v[pl.ds(i * 16, 16)] = jnp.ones((16,), jnp.float32)
        return 0

    lax.fori_loop(0, (CW * 128) // 16, _fill_o, 0)

    pltpu.sync_copy(zb_v, hist_s.at[pl.ds(sid * HIST_PW, HIST_PW)])
    plsc.subcore_barrier()

    for r in range(R):
        pltpu.sync_copy(sidx_hbm.at[cid, r, sid], idx_v)
        for c in range(NCHUNK):
            pltpu.sync_copy(ones_v, hist_s.at[idx_v.at[c]], add=True)

    plsc.subcore_barrier()
    pltpu.sync_copy(hist_s.at[pl.ds(sid * HIST_PW, HIST_PW)], zb_v)
    pltpu.sync_copy(zb_v, out_hbm.at[pl.ds(cid * HIST_N + sid * HIST_PW, HIST_PW)])


@functools.cache
def _deg_kernel():
    return pl.kernel(
        _deg_body,
        out_type=jax.ShapeDtypeStruct((NC * HIST_N,), jnp.float32),
        mesh=_sc_mesh(),
        compiler_params=pltpu.CompilerParams(use_tc_tiling_on_sc=False),
        scratch_types=[
            pltpu.VMEM((NCHUNK, CW * 128), jnp.int32),
            pltpu.VMEM((CW * 128,), jnp.float32),
            pltpu.VMEM((HIST_PW,), jnp.float32),
            pltpu.VMEM_SHARED((HIST_N,), jnp.float32),
        ],
    )


NTOT = R * NCHUNK


def _agg_body(tab_hbm, gidx_hbm, sidx_hbm, out_hbm,
              gi_v, si_v, rows_a, rows_b, zb_v, db_v, agg_s, sem_a, sem_b, gsem):
    cid = lax.axis_index("c")
    sid = lax.axis_index("s")

    def _fill_z(i, _):
        zb_v[i // 4, pl.ds((i % 4) * 16, 16)] = jnp.zeros((16,), jnp.float32)
        return 0

    lax.fori_loop(0, ZROWS * 4, _fill_z, 0)

    for p in range(NPASS):
        for c in range(PER_W // ZROWS):
            pltpu.sync_copy(zb_v, agg_s.at[pl.ds(sid * PER_W + c * ZROWS, ZROWS)])
        plsc.subcore_barrier()

        for r in range(2 * p, min(2 * p + 2, R)):
            pltpu.sync_copy(gidx_hbm.at[cid, r, sid], gi_v)
            pltpu.sync_copy(sidx_hbm.at[cid, r, sid], si_v)

            bufs = (rows_a, rows_b)
            sems = (sem_a, sem_b)
            pend = [None, None]
            for j in range(NCHUNK):
                b = j % 2
                if pend[b] is not None:
                    pend[b].wait()
                pltpu.async_copy(tab_hbm.at[gi_v.at[j]], bufs[b], gsem).wait()
                pend[b] = pltpu.async_copy(bufs[b], agg_s.at[si_v.at[j]],
                                           sems[b], add=True)
            pend[0].wait()
            pend[1].wait()

        plsc.subcore_barrier()
        for c in range(PER_W // ZROWS):
            row = sid * PER_W + c * ZROWS
            pltpu.sync_copy(agg_s.at[pl.ds(row, ZROWS)], db_v)
            pltpu.sync_copy(
                db_v,
                out_hbm.at[pl.ds((cid * NPASS + p) * AGG_ROWS + row, ZROWS)])
        if p < NPASS - 1:
            plsc.subcore_barrier()


@functools.cache
def _agg_kernel():
    return pl.kernel(
        _agg_body,
        out_type=jax.ShapeDtypeStruct((NC * NPASS * AGG_ROWS, M), jnp.float32),
        mesh=_sc_mesh(),
        compiler_params=pltpu.CompilerParams(use_tc_tiling_on_sc=False),
        scratch_types=[
            pltpu.VMEM((NCHUNK, CW * 128), jnp.int32),
            pltpu.VMEM((NCHUNK, CW * 128), jnp.int32),
            pltpu.VMEM((CW * 128, M), jnp.float32),
            pltpu.VMEM((CW * 128, M), jnp.float32),
            pltpu.VMEM((ZROWS, M), jnp.float32),
            pltpu.VMEM((ZROWS, M), jnp.float32),
            pltpu.VMEM_SHARED((AGG_ROWS, M), jnp.float32),
            pltpu.SemaphoreType.DMA,
            pltpu.SemaphoreType.DMA,
            pltpu.SemaphoreType.DMA,
        ],
    )


def _proj_body(feats_ref, w_ref, deg_ref, out_ref):
    d = deg_ref[0, 0, :N] + deg_ref[0, 0, N:2 * N]
    c = lax.rsqrt(jnp.maximum(d, 1.0))
    out_ref[0] = (
        jnp.dot(feats_ref[0], w_ref[0], preferred_element_type=jnp.float32)
        * c[:, None]
    )


def _project(feats_s, w_all, deg_sw):
    return pl.pallas_call(
        _proj_body,
        grid=(2, R),
        in_specs=[
            pl.BlockSpec((1, N, D_IN), lambda d, r: (d, 0, 0)),
            pl.BlockSpec((1, D_IN, M), lambda d, r: (r, 0, 0)),
            pl.BlockSpec((1, 1, HIST_N), lambda d, r: (d, 0, 0)),
        ],
        out_specs=pl.BlockSpec((1, N, M), lambda d, r: (d * R + r, 0, 0)),
        out_shape=jax.ShapeDtypeStruct((2 * R, N, M), jnp.float32),
    )(feats_s, w_all, deg_sw)


def _out_body(agg_ref, deg_ref, fcw_ref, fcb_ref, out_ref):
    r = pl.program_id(1)
    d = deg_ref[0, 0, :N] + deg_ref[0, 0, N:2 * N]
    c = lax.rsqrt(jnp.maximum(d, 1.0))
    x = jnp.maximum(agg_ref[0, 0] * c[:, None], 0.0)
    y = jnp.dot(x, fcw_ref[...], preferred_element_type=jnp.float32)

    @pl.when(r == 0)
    def _():
        out_ref[0] = y + fcb_ref[...]

    @pl.when(r > 0)
    def _():
        out_ref[0] += y


def _fc_out(agg, deg, fc_W, fc_b):
    return pl.pallas_call(
        _out_body,
        grid=(2, R),
        in_specs=[
            pl.BlockSpec((1, 1, N, M), lambda d, r: (d, r // 2, r % 2, 0)),
            pl.BlockSpec((1, 1, HIST_N), lambda d, r: (d, 0, 0)),
            pl.BlockSpec((M, OUT), lambda d, r: (r, 0)),
            pl.BlockSpec((1, OUT), lambda d, r: (0, 0)),
        ],
        out_specs=pl.BlockSpec((1, N, OUT), lambda d, r: (d, 0, 0)),
        out_shape=jax.ShapeDtypeStruct((2, N, OUT), jnp.float32),
    )(agg, deg, fc_W, fc_b.reshape(1, OUT))


def kernel(user_feat, item_feat, edge_index, W_r, fc_W, fc_b):
    src = edge_index[:, 0, :].astype(jnp.int32)
    dst = edge_index[:, 1, :].astype(jnp.int32)

    npad = E_PAD - E
    lane = jnp.arange(npad, dtype=jnp.int32)
    pad_gath = jnp.broadcast_to(lane % 64, (R, npad))
    pad_scat = jnp.broadcast_to(TRASH + (lane % 16), (R, npad))

    roff = (jnp.arange(R, dtype=jnp.int32) * N)[:, None]
    soff = ((jnp.arange(R, dtype=jnp.int32) % 2) * N)[:, None]
    g0 = jnp.concatenate([dst + roff, pad_gath], axis=1)
    s0 = jnp.concatenate([src + soff, pad_scat], axis=1)
    g1 = jnp.concatenate([src + roff + TAB, pad_gath], axis=1)
    s1 = jnp.concatenate([dst + soff, pad_scat], axis=1)
    gidx = jnp.stack([g0, g1]).reshape(NC, R, NS, NCHUNK, CW * 128)
    sidx = jnp.stack([s0, s1]).reshape(NC, R, NS, NCHUNK, CW * 128)

    deg = _deg_kernel()(sidx).reshape(NC, HIST_N)

    feats_s = jnp.stack([item_feat, user_feat])
    deg_sw = deg[::-1]
    tab = _project(feats_s, W_r, deg_sw.reshape(NC, 1, HIST_N)).reshape(2 * TAB, M)

    agg = _agg_kernel()(tab, gidx, sidx).reshape(NC, NPASS, AGG_ROWS, M)

    out = _fc_out(agg, deg.reshape(NC, 1, HIST_N), fc_W, fc_b)
    return out[0], out[1]
